# Initial kernel scaffold; baseline (speedup 1.0000x reference)
import jax
import jax.numpy as jnp
from jax.experimental import pallas as pl


def kernel(state_sides, move_mask, emb_table, basePowers):
    raise NotImplementedError("write your pallas kernel here")


# speedup vs baseline: 3.6160x; 3.6160x over previous
import functools

import jax
import jax.numpy as jnp
from jax import lax
from jax.experimental import pallas as pl
from jax.experimental.pallas import tpu as pltpu
from jax.experimental.pallas import tpu_sc as plsc

_B = 16384
_RES = 12
_NF = 17
_OFF = 128
_VPAD = 1008

_NC = 2
_NS = 16
_L = 16
_NW = _NC * _NS
_BPW = _B // _NW
_EPW = _BPW * 4
_WPB = _RES * _NF


def _tm_body(emb_ref, bp_ref, out_ref):
    prod = emb_ref[:, :_OFF] * bp_ref[...]
    out_ref[...] = jnp.max(prod, axis=1, keepdims=True)


def _table_max(emb_pad, bp_row):
    out = pl.pallas_call(
        _tm_body,
        out_shape=jax.ShapeDtypeStruct((_VPAD, 1), jnp.float32),
    )(emb_pad, bp_row)
    return out.reshape(_VPAD)


def _sc_body(sl_hbm, msk_hbm, tm_hbm, bp_hbm, idx_hbm,
             sl_v, msk_v, tm_v, bp_v, idx_v):
    wid = lax.axis_index("s") * _NC + lax.axis_index("c")
    pltpu.sync_copy(sl_hbm.at[pl.ds(wid * (_BPW * _WPB), _BPW * _WPB)], sl_v)
    pltpu.sync_copy(msk_hbm.at[pl.ds(wid * _EPW, _EPW)], msk_v)
    pltpu.sync_copy(tm_hbm, tm_v)

    def group(i, carry):
        blane = i * _L + lax.iota(jnp.int32, _L)
        rowb = blane * _WPB
        ai = jnp.full((_L,), -1, jnp.int32)
        for r in range(_RES):
            fr = plsc.load_gather(sl_v, [rowb + r * _NF])
            hit = ((fr + 1.0) == 2.0) & (ai < 0)
            ai = jnp.where(hit, r, ai)
        ai = jnp.where(ai < 0, 0, ai)
        tokbase = rowb + ai * _NF + 13
        best = jnp.full((_L,), -jnp.inf, jnp.float32)
        bi = jnp.zeros((_L,), jnp.int32)
        for j in range(4):
            tj = plsc.load_gather(sl_v, [tokbase + j])
            ti = (tj + 1.0).astype(jnp.int32)
            bpv = plsc.load_gather(tm_v, [ti])
            mj = plsc.load_gather(msk_v, [blane * 4 + j])
            bpm = jnp.where(mj != 0, bpv, -1.0)
            plsc.store_scatter(bp_v, [blane * 4 + j], bpm)
            gt = bpm > best
            best = jnp.where(gt, bpm, best)
            bi = jnp.where(gt, j, bi)
        idx_v[pl.ds(i * _L, _L)] = bi
        return carry

    lax.fori_loop(0, _BPW // _L, group, 0)

    pltpu.sync_copy(bp_v, bp_hbm.at[pl.ds(wid * _EPW, _EPW)])
    pltpu.sync_copy(idx_v, idx_hbm.at[pl.ds(wid * _BPW, _BPW)])


def _sc_call(sl_flat, msk_flat, tm_1d):
    mesh = plsc.VectorSubcoreMesh(core_axis_name="c", subcore_axis_name="s")
    fn = pl.kernel(
        _sc_body,
        out_type=[
            jax.ShapeDtypeStruct((_B * 4,), jnp.float32),
            jax.ShapeDtypeStruct((_B,), jnp.int32),
        ],
        scratch_types=[
            pltpu.VMEM((_BPW * _WPB,), jnp.float32),
            pltpu.VMEM((_EPW,), jnp.int32),
            pltpu.VMEM((_VPAD,), jnp.float32),
            pltpu.VMEM((_EPW,), jnp.float32),
            pltpu.VMEM((_BPW,), jnp.int32),
        ],
        mesh=mesh,
        compiler_params=pltpu.CompilerParams(needs_layout_passes=False),
    )
    return fn(sl_flat, msk_flat, tm_1d)


def kernel(state_sides, move_mask, emb_table, basePowers):
    b = state_sides.shape[0]
    sl = state_sides[:, 0, :, 12:29].reshape(b * _WPB)
    msk = move_mask.astype(jnp.int32).reshape(b * 4)
    emb_pad = jnp.pad(emb_table, ((0, _VPAD - emb_table.shape[0]), (0, 0)))
    tm = _table_max(emb_pad, basePowers.reshape(1, _OFF))
    bp_flat, idx = _sc_call(sl, msk, tm)
    return bp_flat.reshape(b, 4), idx
